# baseline (speedup 1.0000x reference)
import functools

import jax
import jax.numpy as jnp
from jax import lax
from jax.experimental import pallas as pl
from jax.experimental.pallas import tpu as pltpu
from jax.experimental.pallas import tpu_sc as plsc

VOCAB = 1000000
EMBED = 64
HIDDEN = 128
BATCH = 16384

NC, NS = 2, 16
NW = NC * NS
TOTAL_ROWS = 2 * BATCH
ROWS_PER_W = TOTAL_ROWS // NW
CHUNK = 128
NCHUNK = ROWS_PER_W // CHUNK


def _gather_body(table_hbm, idx_hbm, out_hbm, idx_v, rows_v, sem):
    wid = lax.axis_index("s") * NC + lax.axis_index("c")
    base = wid * ROWS_PER_W
    pltpu.sync_copy(idx_hbm.at[wid], idx_v)
    copies = []
    for j in range(NCHUNK):
        copies.append(
            pltpu.async_copy(
                table_hbm.at[idx_v.at[j]],
                rows_v.at[pl.ds(j * CHUNK, CHUNK)],
                sem,
            )
        )
    for c in copies:
        c.wait()
    pltpu.sync_copy(rows_v, out_hbm.at[pl.ds(base, ROWS_PER_W)])


_gather_call = functools.partial(
    pl.kernel,
    out_type=jax.ShapeDtypeStruct((TOTAL_ROWS, EMBED), jnp.float32),
    mesh=plsc.VectorSubcoreMesh(core_axis_name="c", subcore_axis_name="s"),
    scratch_types=[
        pltpu.VMEM((NCHUNK, CHUNK), jnp.int32),
        pltpu.VMEM((ROWS_PER_W, EMBED), jnp.float32),
        pltpu.SemaphoreType.DMA,
    ],
    compiler_params=pltpu.CompilerParams(use_tc_tiling_on_sc=False),
)(_gather_body)


def _mlp_body(x_ref, w1_ref, b1_ref, w2_ref, b2_ref, o_ref):
    x = x_ref[...]
    h = lax.dot_general(
        x, w1_ref[...], (((1,), (1,)), ((), ())),
        preferred_element_type=jnp.float32,
    )
    h = jnp.maximum(h + b1_ref[...], 0.0)
    z = lax.dot_general(
        h, w2_ref[...], (((1,), (0,)), ((), ())),
        preferred_element_type=jnp.float32,
    )
    o_ref[...] = jax.nn.sigmoid(z + b2_ref[0, 0])


def _mlp_call(x, W1, b1, W2, b2):
    blk = 2048
    grid = (BATCH // blk,)
    return pl.pallas_call(
        _mlp_body,
        grid=grid,
        in_specs=[
            pl.BlockSpec((blk, 2 * EMBED), lambda i: (i, 0)),
            pl.BlockSpec((HIDDEN, 2 * EMBED), lambda i: (0, 0)),
            pl.BlockSpec((1, HIDDEN), lambda i: (0, 0)),
            pl.BlockSpec((HIDDEN, 1), lambda i: (0, 0)),
            pl.BlockSpec((1, 1), lambda i: (0, 0)),
        ],
        out_specs=pl.BlockSpec((blk, 1), lambda i: (i, 0)),
        out_shape=jax.ShapeDtypeStruct((BATCH, 1), jnp.float32),
    )(x, W1, b1, W2, b2)


@jax.jit
def kernel(pairs, table, W1, b1, W2, b2):
    idx = pairs.reshape(NW, NCHUNK, CHUNK)
    gathered = _gather_call(table, idx)
    x = gathered.reshape(BATCH, 2 * EMBED)
    return _mlp_call(x, W1, b1.reshape(1, HIDDEN), W2.reshape(HIDDEN, 1),
                     b2.reshape(1, 1))


# speedup vs baseline: 1.1127x; 1.1127x over previous
import functools

import jax
import jax.numpy as jnp
from jax import lax
from jax.experimental import pallas as pl
from jax.experimental.pallas import tpu as pltpu
from jax.experimental.pallas import tpu_sc as plsc

VOCAB = 1000000
EMBED = 64
HIDDEN = 128
BATCH = 16384

NC, NS = 2, 16
NW = NC * NS
TOTAL_ROWS = 2 * BATCH
ROWS_PER_W = TOTAL_ROWS // NW
CHUNK = 128
NCHUNK = ROWS_PER_W // CHUNK
NBUF = 4


def _gather_body(table_hbm, idx_hbm, out_hbm, idx_v, rows_v, gsem, wsem):
    wid = lax.axis_index("s") * NC + lax.axis_index("c")
    base = wid * ROWS_PER_W
    pltpu.sync_copy(idx_hbm.at[wid], idx_v)
    gathers = []
    writes = [None] * NBUF
    for j in range(NBUF):
        gathers.append(
            pltpu.async_copy(table_hbm.at[idx_v.at[j]], rows_v.at[j], gsem)
        )
    for j in range(NCHUNK):
        gathers[j].wait()
        writes[j % NBUF] = pltpu.async_copy(
            rows_v.at[j % NBUF],
            out_hbm.at[pl.ds(base + j * CHUNK, CHUNK)],
            wsem,
        )
        nxt = j + NBUF
        if nxt < NCHUNK:
            writes[nxt % NBUF].wait()
            gathers.append(
                pltpu.async_copy(table_hbm.at[idx_v.at[nxt]],
                                 rows_v.at[nxt % NBUF], gsem)
            )
    for j in range(NBUF):
        writes[(NCHUNK - NBUF + j) % NBUF].wait()


_gather_call = functools.partial(
    pl.kernel,
    out_type=jax.ShapeDtypeStruct((TOTAL_ROWS, HIDDEN), jnp.int32),
    mesh=plsc.VectorSubcoreMesh(core_axis_name="c", subcore_axis_name="s"),
    scratch_types=[
        pltpu.VMEM((NCHUNK, CHUNK), jnp.int32),
        pltpu.VMEM((NBUF, CHUNK, HIDDEN), jnp.int32),
        pltpu.SemaphoreType.DMA,
        pltpu.SemaphoreType.DMA,
    ],
    compiler_params=pltpu.CompilerParams(use_tc_tiling_on_sc=True),
)(_gather_body)


def _select_row(w, b_hi, b_half):
    low = lax.bitcast_convert_type(w << 16, jnp.float32)
    high = lax.bitcast_convert_type(w & jnp.int32(-65536), jnp.float32)
    row = low + b_hi * (high - low)
    return row[:, :EMBED] + b_half * (row[:, EMBED:] - row[:, :EMBED])


def _mlp_body(g0_ref, g1_ref, s0_ref, s1_ref, w1_ref, b1_ref, w2_ref,
              b2_ref, o_ref):
    s0 = s0_ref[...]
    s1 = s1_ref[...]
    x0 = _select_row(g0_ref[...], s0[:, 1:2], s0[:, 0:1])
    x1 = _select_row(g1_ref[...], s1[:, 1:2], s1[:, 0:1])
    w1 = w1_ref[...]
    h = lax.dot_general(
        x0, w1[:, :EMBED], (((1,), (1,)), ((), ())),
        preferred_element_type=jnp.float32,
    )
    h = h + lax.dot_general(
        x1, w1[:, EMBED:], (((1,), (1,)), ((), ())),
        preferred_element_type=jnp.float32,
    )
    h = jnp.maximum(h + b1_ref[...], 0.0)
    z = lax.dot_general(
        h, w2_ref[...], (((1,), (0,)), ((), ())),
        preferred_element_type=jnp.float32,
    )
    o_ref[...] = jax.nn.sigmoid(z + b2_ref[0, 0])


def _mlp_call(g, sub, W1, b1, W2, b2):
    blk = 2048
    nblk = BATCH // blk
    return pl.pallas_call(
        _mlp_body,
        grid=(nblk,),
        in_specs=[
            pl.BlockSpec((blk, HIDDEN), lambda i: (i, 0)),
            pl.BlockSpec((blk, HIDDEN), lambda i: (nblk + i, 0)),
            pl.BlockSpec((blk, 2), lambda i: (i, 0)),
            pl.BlockSpec((blk, 2), lambda i: (nblk + i, 0)),
            pl.BlockSpec((HIDDEN, HIDDEN), lambda i: (0, 0)),
            pl.BlockSpec((1, HIDDEN), lambda i: (0, 0)),
            pl.BlockSpec((HIDDEN, 1), lambda i: (0, 0)),
            pl.BlockSpec((1, 1), lambda i: (0, 0)),
        ],
        out_specs=pl.BlockSpec((blk, 1), lambda i: (i, 0)),
        out_shape=jax.ShapeDtypeStruct((BATCH, 1), jnp.float32),
    )(g, g, sub, sub, W1, b1, W2, b2)


def _bf16bits(x):
    b = lax.bitcast_convert_type(x, jnp.int32)
    return ((b + jnp.int32(0x7FFF) + ((b >> 16) & jnp.int32(1))) >> 16) \
        & jnp.int32(0xFFFF)


VBLK = 2048


def _pack_body(tt_ref, o_ref):
    xt = lax.transpose(tt_ref[...], (1, 0))
    v4 = xt.reshape(VBLK // 4, 4, EMBED)
    lo = _bf16bits(jnp.concatenate([v4[:, 0, :], v4[:, 1, :]], axis=1))
    hi = _bf16bits(jnp.concatenate([v4[:, 2, :], v4[:, 3, :]], axis=1))
    o_ref[...] = lo | (hi << 16)


def _pack_call(tt):
    return pl.pallas_call(
        _pack_body,
        grid=(VOCAB // VBLK,),
        in_specs=[pl.BlockSpec((EMBED, VBLK), lambda i: (0, i))],
        out_specs=pl.BlockSpec((VBLK // 4, 2 * EMBED), lambda i: (i, 0)),
        out_shape=jax.ShapeDtypeStruct((VOCAB // 4, 2 * EMBED), jnp.int32),
    )(tt)


@jax.jit
def kernel(pairs, table, W1, b1, W2, b2):
    t32 = _pack_call(table.T)
    flat = pairs.T.reshape(TOTAL_ROWS)
    idx = (flat >> 2).reshape(NW, NCHUNK, CHUNK)
    sub = jnp.stack(
        [(flat & 1).astype(jnp.float32),
         ((flat >> 1) & 1).astype(jnp.float32)], axis=1)
    g = _gather_call(t32, idx)
    return _mlp_call(g, sub, W1, b1.reshape(1, HIDDEN),
                     W2.reshape(HIDDEN, 1), b2.reshape(1, 1))


# speedup vs baseline: 1.2416x; 1.1158x over previous
import functools

import jax
import jax.numpy as jnp
from jax import lax
from jax.experimental import pallas as pl
from jax.experimental.pallas import tpu as pltpu
from jax.experimental.pallas import tpu_sc as plsc

VOCAB = 1000000
EMBED = 64
HIDDEN = 128
BATCH = 16384

NC, NS = 2, 16
NW = NC * NS
TOTAL_ROWS = 2 * BATCH
ROWS_PER_W = TOTAL_ROWS // NW
CHUNK = 128
NCHUNK = ROWS_PER_W // CHUNK
NBUF = 4


def _gather_body(table_hbm, idx_hbm, out_hbm, idx_v, rows_v, gsem, wsem):
    wid = lax.axis_index("s") * NC + lax.axis_index("c")
    base = wid * ROWS_PER_W
    pltpu.sync_copy(idx_hbm.at[wid], idx_v)
    gathers = []
    writes = [None] * NBUF
    for j in range(NBUF):
        gathers.append(
            pltpu.async_copy(table_hbm.at[idx_v.at[j]], rows_v.at[j], gsem)
        )
    for j in range(NCHUNK):
        gathers[j].wait()
        writes[j % NBUF] = pltpu.async_copy(
            rows_v.at[j % NBUF],
            out_hbm.at[pl.ds(base + j * CHUNK, CHUNK)],
            wsem,
        )
        nxt = j + NBUF
        if nxt < NCHUNK:
            writes[nxt % NBUF].wait()
            gathers.append(
                pltpu.async_copy(table_hbm.at[idx_v.at[nxt]],
                                 rows_v.at[nxt % NBUF], gsem)
            )
    for j in range(NBUF):
        writes[(NCHUNK - NBUF + j) % NBUF].wait()


_gather_call = functools.partial(
    pl.kernel,
    out_type=jax.ShapeDtypeStruct((TOTAL_ROWS, HIDDEN), jnp.int32),
    mesh=plsc.VectorSubcoreMesh(core_axis_name="c", subcore_axis_name="s"),
    scratch_types=[
        pltpu.VMEM((NCHUNK, CHUNK), jnp.int32),
        pltpu.VMEM((NBUF, CHUNK, HIDDEN), jnp.int32),
        pltpu.SemaphoreType.DMA,
        pltpu.SemaphoreType.DMA,
    ],
    compiler_params=pltpu.CompilerParams(use_tc_tiling_on_sc=True),
)(_gather_body)


def _select_row(w, b_hi, b_half):
    low = lax.bitcast_convert_type(w << 16, jnp.float32)
    high = lax.bitcast_convert_type(w & jnp.int32(-65536), jnp.float32)
    row = low + b_hi * (high - low)
    return row[:, :EMBED] + b_half * (row[:, EMBED:] - row[:, :EMBED])


def _mlp_body(g0_ref, g1_ref, s0_ref, s1_ref, w1_ref, b1_ref, w2_ref,
              b2_ref, o_ref):
    s0 = s0_ref[...]
    s1 = s1_ref[...]
    x0 = _select_row(g0_ref[...], s0[:, 1:2], s0[:, 0:1])
    x1 = _select_row(g1_ref[...], s1[:, 1:2], s1[:, 0:1])
    w1 = w1_ref[...]
    h = lax.dot_general(
        x0, w1[:, :EMBED], (((1,), (1,)), ((), ())),
        preferred_element_type=jnp.float32,
    )
    h = h + lax.dot_general(
        x1, w1[:, EMBED:], (((1,), (1,)), ((), ())),
        preferred_element_type=jnp.float32,
    )
    h = jnp.maximum(h + b1_ref[...], 0.0)
    z = lax.dot_general(
        h, w2_ref[...], (((1,), (0,)), ((), ())),
        preferred_element_type=jnp.float32,
    )
    o_ref[...] = jax.nn.sigmoid(z + b2_ref[0, 0])


def _mlp_call(g, sub, W1, b1, W2, b2):
    blk = 2048
    nblk = BATCH // blk
    return pl.pallas_call(
        _mlp_body,
        grid=(nblk,),
        in_specs=[
            pl.BlockSpec((blk, HIDDEN), lambda i: (i, 0)),
            pl.BlockSpec((blk, HIDDEN), lambda i: (nblk + i, 0)),
            pl.BlockSpec((blk, 2), lambda i: (i, 0)),
            pl.BlockSpec((blk, 2), lambda i: (nblk + i, 0)),
            pl.BlockSpec((HIDDEN, HIDDEN), lambda i: (0, 0)),
            pl.BlockSpec((1, HIDDEN), lambda i: (0, 0)),
            pl.BlockSpec((HIDDEN, 1), lambda i: (0, 0)),
            pl.BlockSpec((1, 1), lambda i: (0, 0)),
        ],
        out_specs=pl.BlockSpec((blk, 1), lambda i: (i, 0)),
        out_shape=jax.ShapeDtypeStruct((BATCH, 1), jnp.float32),
    )(g, g, sub, sub, W1, b1, W2, b2)


def _bf16bits(x):
    b = lax.bitcast_convert_type(x, jnp.int32)
    return ((b + jnp.int32(0x7FFF) + ((b >> 16) & jnp.int32(1))) >> 16) \
        & jnp.int32(0xFFFF)


VBLK = 2048


def _pack_body(tt_ref, o_ref):
    eye = jnp.float32(
        lax.broadcasted_iota(jnp.int32, (EMBED, EMBED), 0)
        == lax.broadcasted_iota(jnp.int32, (EMBED, EMBED), 1))
    xt = lax.dot_general(
        tt_ref[...], eye, (((0,), (0,)), ((), ())),
        preferred_element_type=jnp.float32)
    q4 = VBLK // 4
    lo = _bf16bits(jnp.concatenate([xt[:q4], xt[q4:2 * q4]], axis=1))
    hi = _bf16bits(jnp.concatenate([xt[2 * q4:3 * q4], xt[3 * q4:]], axis=1))
    o_ref[...] = lo | (hi << 16)


def _pack_call(tt):
    return pl.pallas_call(
        _pack_body,
        grid=(VOCAB // VBLK,),
        in_specs=[pl.BlockSpec((EMBED, VBLK), lambda i: (0, i))],
        out_specs=pl.BlockSpec((VBLK // 4, 2 * EMBED), lambda i: (i, 0)),
        out_shape=jax.ShapeDtypeStruct((VOCAB // 4, 2 * EMBED), jnp.int32),
    )(tt)


@jax.jit
def kernel(pairs, table, W1, b1, W2, b2):
    t32 = _pack_call(table.T)
    flat = pairs.T.reshape(TOTAL_ROWS)
    q = (flat % VBLK) // (VBLK // 4)
    row = (flat // VBLK) * (VBLK // 4) + flat % (VBLK // 4)
    idx = row.reshape(NW, NCHUNK, CHUNK)
    sub = jnp.stack(
        [(q & 1).astype(jnp.float32),
         (q >> 1).astype(jnp.float32)], axis=1)
    g = _gather_call(t32, idx)
    return _mlp_call(g, sub, W1, b1.reshape(1, HIDDEN),
                     W2.reshape(HIDDEN, 1), b2.reshape(1, 1))


# speedup vs baseline: 2.0919x; 1.6849x over previous
import functools

import jax
import jax.numpy as jnp
from jax import lax
from jax.experimental import pallas as pl
from jax.experimental.pallas import tpu as pltpu
from jax.experimental.pallas import tpu_sc as plsc

VOCAB = 1000000
EMBED = 64
HIDDEN = 128
BATCH = 16384

NC, NS = 2, 16
NW = NC * NS
TOTAL_ROWS = 2 * BATCH
ROWS_PER_W = TOTAL_ROWS // NW
CHUNK = 128
NCHUNK = ROWS_PER_W // CHUNK
NBUF = 4


def _gather_body(table_hbm, idx_hbm, out_hbm, idx_v, rows_v, gsem, wsem):
    wid = lax.axis_index("s") * NC + lax.axis_index("c")
    base = wid * ROWS_PER_W
    pltpu.sync_copy(idx_hbm.at[wid], idx_v)
    gathers = []
    writes = [None] * NBUF
    for j in range(NBUF):
        gathers.append(
            pltpu.async_copy(table_hbm.at[idx_v.at[j]], rows_v.at[j], gsem)
        )
    for j in range(NCHUNK):
        gathers[j].wait()
        writes[j % NBUF] = pltpu.async_copy(
            rows_v.at[j % NBUF],
            out_hbm.at[pl.ds(base + j * CHUNK, CHUNK)],
            wsem,
        )
        nxt = j + NBUF
        if nxt < NCHUNK:
            writes[nxt % NBUF].wait()
            gathers.append(
                pltpu.async_copy(table_hbm.at[idx_v.at[nxt]],
                                 rows_v.at[nxt % NBUF], gsem)
            )
    for j in range(NBUF):
        writes[(NCHUNK - NBUF + j) % NBUF].wait()


_gather_call = functools.partial(
    pl.kernel,
    out_type=jax.ShapeDtypeStruct((TOTAL_ROWS, HIDDEN), jnp.int32),
    mesh=plsc.VectorSubcoreMesh(core_axis_name="c", subcore_axis_name="s"),
    scratch_types=[
        pltpu.VMEM((NCHUNK, CHUNK), jnp.int32),
        pltpu.VMEM((NBUF, CHUNK, HIDDEN), jnp.int32),
        pltpu.SemaphoreType.DMA,
        pltpu.SemaphoreType.DMA,
    ],
    compiler_params=pltpu.CompilerParams(use_tc_tiling_on_sc=True),
)(_gather_body)


def _select_row(w, b_hi, b_half):
    low = lax.bitcast_convert_type(w << 16, jnp.float32)
    high = lax.bitcast_convert_type(w & jnp.int32(-65536), jnp.float32)
    row = low + b_hi * (high - low)
    return row[:, :EMBED] + b_half * (row[:, EMBED:] - row[:, :EMBED])


def _mlp_body(g0_ref, g1_ref, s0_ref, s1_ref, w1_ref, b1_ref, w2_ref,
              b2_ref, o_ref):
    s0 = s0_ref[...]
    s1 = s1_ref[...]
    x0 = _select_row(g0_ref[...], s0[:, 1:2], s0[:, 0:1])
    x1 = _select_row(g1_ref[...], s1[:, 1:2], s1[:, 0:1])
    w1 = w1_ref[...]
    h = lax.dot_general(
        x0, w1[:, :EMBED], (((1,), (1,)), ((), ())),
        preferred_element_type=jnp.float32,
    )
    h = h + lax.dot_general(
        x1, w1[:, EMBED:], (((1,), (1,)), ((), ())),
        preferred_element_type=jnp.float32,
    )
    h = jnp.maximum(h + b1_ref[...], 0.0)
    z = lax.dot_general(
        h, w2_ref[...], (((1,), (0,)), ((), ())),
        preferred_element_type=jnp.float32,
    )
    o_ref[...] = jax.nn.sigmoid(z + b2_ref[0, 0])


def _mlp_call(g, sub, W1, b1, W2, b2):
    blk = 2048
    nblk = BATCH // blk
    return pl.pallas_call(
        _mlp_body,
        grid=(nblk,),
        in_specs=[
            pl.BlockSpec((blk, HIDDEN), lambda i: (i, 0)),
            pl.BlockSpec((blk, HIDDEN), lambda i: (nblk + i, 0)),
            pl.BlockSpec((blk, 2), lambda i: (i, 0)),
            pl.BlockSpec((blk, 2), lambda i: (nblk + i, 0)),
            pl.BlockSpec((HIDDEN, HIDDEN), lambda i: (0, 0)),
            pl.BlockSpec((1, HIDDEN), lambda i: (0, 0)),
            pl.BlockSpec((HIDDEN, 1), lambda i: (0, 0)),
            pl.BlockSpec((1, 1), lambda i: (0, 0)),
        ],
        out_specs=pl.BlockSpec((blk, 1), lambda i: (i, 0)),
        out_shape=jax.ShapeDtypeStruct((BATCH, 1), jnp.float32),
    )(g, g, sub, sub, W1, b1, W2, b2)


def _bf16bits(x):
    b = lax.bitcast_convert_type(x, jnp.int32)
    return ((b + jnp.int32(0x7FFF) + ((b >> 16) & jnp.int32(1))) >> 16) \
        & jnp.int32(0xFFFF)


VBLK = 8192


def _pack_body(tt_ref, o_ref):
    eye = jnp.float32(
        lax.broadcasted_iota(jnp.int32, (EMBED, EMBED), 0)
        == lax.broadcasted_iota(jnp.int32, (EMBED, EMBED), 1))
    xt = lax.dot_general(
        tt_ref[...], eye, (((0,), (0,)), ((), ())),
        preferred_element_type=jnp.float32)
    q4 = VBLK // 4
    lo = _bf16bits(jnp.concatenate([xt[:q4], xt[q4:2 * q4]], axis=1))
    hi = _bf16bits(jnp.concatenate([xt[2 * q4:3 * q4], xt[3 * q4:]], axis=1))
    o_ref[...] = lo | (hi << 16)


NPACK = -(-VOCAB // VBLK)


def _pack_call(tt):
    return pl.pallas_call(
        _pack_body,
        grid=(NPACK,),
        in_specs=[pl.BlockSpec((EMBED, VBLK), lambda i: (0, i))],
        out_specs=pl.BlockSpec((VBLK // 4, 2 * EMBED), lambda i: (i, 0)),
        out_shape=jax.ShapeDtypeStruct((NPACK * VBLK // 4, 2 * EMBED),
                                       jnp.int32),
    )(tt)


@jax.jit
def kernel(pairs, table, W1, b1, W2, b2):
    t32 = _pack_call(table.T)
    flat = pairs.T.reshape(TOTAL_ROWS)
    q = (flat % VBLK) // (VBLK // 4)
    row = (flat // VBLK) * (VBLK // 4) + flat % (VBLK // 4)
    idx = row.reshape(NW, NCHUNK, CHUNK)
    sub = jnp.stack(
        [(q & 1).astype(jnp.float32),
         (q >> 1).astype(jnp.float32)], axis=1)
    g = _gather_call(t32, idx)
    return _mlp_call(g, sub, W1, b1.reshape(1, HIDDEN),
                     W2.reshape(HIDDEN, 1), b2.reshape(1, 1))


# speedup vs baseline: 2.3564x; 1.1264x over previous
import functools

import jax
import jax.numpy as jnp
from jax import lax
from jax.experimental import pallas as pl
from jax.experimental.pallas import tpu as pltpu
from jax.experimental.pallas import tpu_sc as plsc

VOCAB = 1000000
EMBED = 64
HIDDEN = 128
BATCH = 16384

NC, NS = 2, 16
NW = NC * NS
TOTAL_ROWS = 2 * BATCH
ROWS_PER_W = TOTAL_ROWS // NW
CHUNK = 128
NCHUNK = ROWS_PER_W // CHUNK
NBUF = 4


def _gather_body(table_hbm, idx_hbm, out_hbm, idx_v, rows_v, gsem, wsem):
    wid = lax.axis_index("s") * NC + lax.axis_index("c")
    base = wid * ROWS_PER_W
    pltpu.sync_copy(idx_hbm.at[wid], idx_v)
    gathers = []
    writes = [None] * NBUF
    for j in range(NBUF):
        gathers.append(
            pltpu.async_copy(table_hbm.at[idx_v.at[j]], rows_v.at[j], gsem)
        )
    for j in range(NCHUNK):
        gathers[j].wait()
        writes[j % NBUF] = pltpu.async_copy(
            rows_v.at[j % NBUF],
            out_hbm.at[pl.ds(base + j * CHUNK, CHUNK)],
            wsem,
        )
        nxt = j + NBUF
        if nxt < NCHUNK:
            writes[nxt % NBUF].wait()
            gathers.append(
                pltpu.async_copy(table_hbm.at[idx_v.at[nxt]],
                                 rows_v.at[nxt % NBUF], gsem)
            )
    for j in range(NBUF):
        writes[(NCHUNK - NBUF + j) % NBUF].wait()


_gather_call = functools.partial(
    pl.kernel,
    out_type=jax.ShapeDtypeStruct((TOTAL_ROWS, HIDDEN), jnp.int32),
    mesh=plsc.VectorSubcoreMesh(core_axis_name="c", subcore_axis_name="s"),
    scratch_types=[
        pltpu.VMEM((NCHUNK, CHUNK), jnp.int32),
        pltpu.VMEM((NBUF, CHUNK, HIDDEN), jnp.int32),
        pltpu.SemaphoreType.DMA,
        pltpu.SemaphoreType.DMA,
    ],
    compiler_params=pltpu.CompilerParams(use_tc_tiling_on_sc=True),
)(_gather_body)


def _select_row(w, b_hi, b_half):
    low = lax.bitcast_convert_type(w << 16, jnp.float32)
    high = lax.bitcast_convert_type(w & jnp.int32(-65536), jnp.float32)
    row = low + b_hi * (high - low)
    return row[:, :EMBED] + b_half * (row[:, EMBED:] - row[:, :EMBED])


def _mlp_body(g0_ref, g1_ref, s0_ref, s1_ref, w1_ref, b1_ref, w2_ref,
              b2_ref, o_ref):
    s0 = s0_ref[...]
    s1 = s1_ref[...]
    x0 = _select_row(g0_ref[...], s0[:, 1:2], s0[:, 0:1])
    x1 = _select_row(g1_ref[...], s1[:, 1:2], s1[:, 0:1])
    w1 = w1_ref[...]
    h = lax.dot_general(
        x0, w1[:, :EMBED], (((1,), (1,)), ((), ())),
        preferred_element_type=jnp.float32,
    )
    h = h + lax.dot_general(
        x1, w1[:, EMBED:], (((1,), (1,)), ((), ())),
        preferred_element_type=jnp.float32,
    )
    h = jnp.maximum(h + b1_ref[...], 0.0)
    z = lax.dot_general(
        h, w2_ref[...], (((1,), (0,)), ((), ())),
        preferred_element_type=jnp.float32,
    )
    o_ref[...] = jax.nn.sigmoid(z + b2_ref[0, 0])


def _mlp_call(g, sub, W1, b1, W2, b2):
    blk = 2048
    nblk = BATCH // blk
    return pl.pallas_call(
        _mlp_body,
        grid=(nblk,),
        in_specs=[
            pl.BlockSpec((blk, HIDDEN), lambda i: (i, 0)),
            pl.BlockSpec((blk, HIDDEN), lambda i: (nblk + i, 0)),
            pl.BlockSpec((blk, 2), lambda i: (i, 0)),
            pl.BlockSpec((blk, 2), lambda i: (nblk + i, 0)),
            pl.BlockSpec((HIDDEN, HIDDEN), lambda i: (0, 0)),
            pl.BlockSpec((1, HIDDEN), lambda i: (0, 0)),
            pl.BlockSpec((HIDDEN, 1), lambda i: (0, 0)),
            pl.BlockSpec((1, 1), lambda i: (0, 0)),
        ],
        out_specs=pl.BlockSpec((blk, 1), lambda i: (i, 0)),
        out_shape=jax.ShapeDtypeStruct((BATCH, 1), jnp.float32),
    )(g, g, sub, sub, W1, b1, W2, b2)


def _bf16bits(x):
    b = lax.bitcast_convert_type(x, jnp.int32)
    return ((b + jnp.int32(0x7FFF) + ((b >> 16) & jnp.int32(1))) >> 16) \
        & jnp.int32(0xFFFF)


VBLK = 8192


def _pack_body(tt_ref, o_ref):
    eye = jnp.bfloat16(
        lax.broadcasted_iota(jnp.int32, (EMBED, EMBED), 0)
        == lax.broadcasted_iota(jnp.int32, (EMBED, EMBED), 1))
    xt = lax.dot_general(
        tt_ref[...].astype(jnp.bfloat16), eye, (((0,), (0,)), ((), ())),
        preferred_element_type=jnp.float32)
    q4 = VBLK // 4
    lo = lax.bitcast_convert_type(
        jnp.concatenate([xt[:q4], xt[q4:2 * q4]], axis=1), jnp.int32)
    hi = lax.bitcast_convert_type(
        jnp.concatenate([xt[2 * q4:3 * q4], xt[3 * q4:]], axis=1), jnp.int32)
    o_ref[...] = ((lo >> 16) & jnp.int32(0xFFFF)) | (hi & jnp.int32(-65536))


NPACK = -(-VOCAB // VBLK)


def _pack_call(tt):
    return pl.pallas_call(
        _pack_body,
        grid=(NPACK,),
        in_specs=[pl.BlockSpec((EMBED, VBLK), lambda i: (0, i))],
        out_specs=pl.BlockSpec((VBLK // 4, 2 * EMBED), lambda i: (i, 0)),
        out_shape=jax.ShapeDtypeStruct((NPACK * VBLK // 4, 2 * EMBED),
                                       jnp.int32),
    )(tt)


@jax.jit
def kernel(pairs, table, W1, b1, W2, b2):
    t32 = _pack_call(table.T)
    flat = pairs.T.reshape(TOTAL_ROWS)
    q = (flat % VBLK) // (VBLK // 4)
    row = (flat // VBLK) * (VBLK // 4) + flat % (VBLK // 4)
    idx = row.reshape(NW, NCHUNK, CHUNK)
    sub = jnp.stack(
        [(q & 1).astype(jnp.float32),
         (q >> 1).astype(jnp.float32)], axis=1)
    g = _gather_call(t32, idx)
    return _mlp_call(g, sub, W1, b1.reshape(1, HIDDEN),
                     W2.reshape(HIDDEN, 1), b2.reshape(1, 1))


# speedup vs baseline: 2.7209x; 1.1547x over previous
import functools

import jax
import jax.numpy as jnp
from jax import lax
from jax.experimental import pallas as pl
from jax.experimental.pallas import tpu as pltpu
from jax.experimental.pallas import tpu_sc as plsc

VOCAB = 1000000
EMBED = 64
HIDDEN = 128
BATCH = 16384

NC, NS = 2, 16
NW = NC * NS
TOTAL_ROWS = 2 * BATCH
ROWS_PER_W = TOTAL_ROWS // NW
CHUNK = 128
NCHUNK = ROWS_PER_W // CHUNK
NBUF = 4


def _gather_body(table_hbm, idx_hbm, out_hbm, idx_v, rows_v, gsem, wsem):
    wid = lax.axis_index("s") * NC + lax.axis_index("c")
    base = wid * ROWS_PER_W
    pltpu.sync_copy(idx_hbm.at[wid], idx_v)
    gathers = []
    writes = [None] * NBUF
    for j in range(NBUF):
        gathers.append(
            pltpu.async_copy(table_hbm.at[idx_v.at[j]], rows_v.at[j], gsem)
        )
    for j in range(NCHUNK):
        gathers[j].wait()
        writes[j % NBUF] = pltpu.async_copy(
            rows_v.at[j % NBUF],
            out_hbm.at[pl.ds(base + j * CHUNK, CHUNK)],
            wsem,
        )
        nxt = j + NBUF
        if nxt < NCHUNK:
            writes[nxt % NBUF].wait()
            gathers.append(
                pltpu.async_copy(table_hbm.at[idx_v.at[nxt]],
                                 rows_v.at[nxt % NBUF], gsem)
            )
    for j in range(NBUF):
        writes[(NCHUNK - NBUF + j) % NBUF].wait()


_gather_call = functools.partial(
    pl.kernel,
    out_type=jax.ShapeDtypeStruct((TOTAL_ROWS, HIDDEN), jnp.int32),
    mesh=plsc.VectorSubcoreMesh(core_axis_name="c", subcore_axis_name="s"),
    scratch_types=[
        pltpu.VMEM((NCHUNK, CHUNK), jnp.int32),
        pltpu.VMEM((NBUF, CHUNK, HIDDEN), jnp.int32),
        pltpu.SemaphoreType.DMA,
        pltpu.SemaphoreType.DMA,
    ],
    compiler_params=pltpu.CompilerParams(use_tc_tiling_on_sc=True),
)(_gather_body)


def _select_row(w, b_hi, b_half):
    low = lax.bitcast_convert_type(w << 16, jnp.float32)
    high = lax.bitcast_convert_type(w & jnp.int32(-65536), jnp.float32)
    row = low + b_hi * (high - low)
    return row[:, :EMBED] + b_half * (row[:, EMBED:] - row[:, :EMBED])


def _mlp_body(g0_ref, g1_ref, s0_ref, s1_ref, w1_ref, b1_ref, w2_ref,
              b2_ref, o_ref):
    s0 = s0_ref[...]
    s1 = s1_ref[...]
    x0 = _select_row(g0_ref[...], s0[:, 1:2], s0[:, 0:1])
    x1 = _select_row(g1_ref[...], s1[:, 1:2], s1[:, 0:1])
    w1 = w1_ref[...]
    h = lax.dot_general(
        x0, w1[:, :EMBED], (((1,), (1,)), ((), ())),
        preferred_element_type=jnp.float32,
    )
    h = h + lax.dot_general(
        x1, w1[:, EMBED:], (((1,), (1,)), ((), ())),
        preferred_element_type=jnp.float32,
    )
    h = jnp.maximum(h + b1_ref[...], 0.0)
    z = lax.dot_general(
        h, w2_ref[...], (((1,), (0,)), ((), ())),
        preferred_element_type=jnp.float32,
    )
    o_ref[...] = jax.nn.sigmoid(z + b2_ref[0, 0])


def _mlp_call(g, sub, W1, b1, W2, b2):
    blk = 4096
    nblk = BATCH // blk
    return pl.pallas_call(
        _mlp_body,
        grid=(nblk,),
        in_specs=[
            pl.BlockSpec((blk, HIDDEN), lambda i: (i, 0)),
            pl.BlockSpec((blk, HIDDEN), lambda i: (nblk + i, 0)),
            pl.BlockSpec((blk, 2), lambda i: (i, 0)),
            pl.BlockSpec((blk, 2), lambda i: (nblk + i, 0)),
            pl.BlockSpec((HIDDEN, HIDDEN), lambda i: (0, 0)),
            pl.BlockSpec((1, HIDDEN), lambda i: (0, 0)),
            pl.BlockSpec((HIDDEN, 1), lambda i: (0, 0)),
            pl.BlockSpec((1, 1), lambda i: (0, 0)),
        ],
        out_specs=pl.BlockSpec((blk, 1), lambda i: (i, 0)),
        out_shape=jax.ShapeDtypeStruct((BATCH, 1), jnp.float32),
    )(g, g, sub, sub, W1, b1, W2, b2)


def _bf16bits(x):
    b = lax.bitcast_convert_type(x, jnp.int32)
    return ((b + jnp.int32(0x7FFF) + ((b >> 16) & jnp.int32(1))) >> 16) \
        & jnp.int32(0xFFFF)


VBLK = 16384


def _pack_body(tt_ref, o_ref):
    eye = jnp.bfloat16(
        lax.broadcasted_iota(jnp.int32, (EMBED, EMBED), 0)
        == lax.broadcasted_iota(jnp.int32, (EMBED, EMBED), 1))
    xt = lax.dot_general(
        tt_ref[...].astype(jnp.bfloat16), eye, (((0,), (0,)), ((), ())),
        preferred_element_type=jnp.float32)
    q4 = VBLK // 4
    lo = lax.bitcast_convert_type(
        jnp.concatenate([xt[:q4], xt[q4:2 * q4]], axis=1), jnp.int32)
    hi = lax.bitcast_convert_type(
        jnp.concatenate([xt[2 * q4:3 * q4], xt[3 * q4:]], axis=1), jnp.int32)
    o_ref[...] = ((lo >> 16) & jnp.int32(0xFFFF)) | (hi & jnp.int32(-65536))


NPACK = -(-VOCAB // VBLK)


def _pack_call(tt):
    return pl.pallas_call(
        _pack_body,
        grid=(NPACK,),
        in_specs=[pl.BlockSpec((EMBED, VBLK), lambda i: (0, i))],
        out_specs=pl.BlockSpec((VBLK // 4, 2 * EMBED), lambda i: (i, 0)),
        out_shape=jax.ShapeDtypeStruct((NPACK * VBLK // 4, 2 * EMBED),
                                       jnp.int32),
    )(tt)


@jax.jit
def kernel(pairs, table, W1, b1, W2, b2):
    t32 = _pack_call(table.T)
    flat = pairs.T.reshape(TOTAL_ROWS)
    q = (flat % VBLK) // (VBLK // 4)
    row = (flat // VBLK) * (VBLK // 4) + flat % (VBLK // 4)
    idx = row.reshape(NW, NCHUNK, CHUNK)
    sub = jnp.stack(
        [(q & 1).astype(jnp.float32),
         (q >> 1).astype(jnp.float32)], axis=1)
    g = _gather_call(t32, idx)
    return _mlp_call(g, sub, W1, b1.reshape(1, HIDDEN),
                     W2.reshape(HIDDEN, 1), b2.reshape(1, 1))


# speedup vs baseline: 2.9234x; 1.0744x over previous
import functools

import jax
import jax.numpy as jnp
from jax import lax
from jax.experimental import pallas as pl
from jax.experimental.pallas import tpu as pltpu
from jax.experimental.pallas import tpu_sc as plsc

VOCAB = 1000000
EMBED = 64
HIDDEN = 128
BATCH = 16384

NC, NS = 2, 16
NW = NC * NS
TOTAL_ROWS = 2 * BATCH
ROWS_PER_W = TOTAL_ROWS // NW
CHUNK = 128
NCHUNK = ROWS_PER_W // CHUNK
NBUF = 4


def _gather_body(table_hbm, idx_hbm, out_hbm, idx_v, rows_v, gsem, wsem):
    wid = lax.axis_index("s") * NC + lax.axis_index("c")
    base = wid * ROWS_PER_W
    pltpu.sync_copy(idx_hbm.at[wid], idx_v)
    gathers = []
    writes = [None] * NBUF
    for j in range(NBUF):
        gathers.append(
            pltpu.async_copy(table_hbm.at[idx_v.at[j]], rows_v.at[j], gsem)
        )
    for j in range(NCHUNK):
        gathers[j].wait()
        writes[j % NBUF] = pltpu.async_copy(
            rows_v.at[j % NBUF],
            out_hbm.at[pl.ds(base + j * CHUNK, CHUNK)],
            wsem,
        )
        nxt = j + NBUF
        if nxt < NCHUNK:
            writes[nxt % NBUF].wait()
            gathers.append(
                pltpu.async_copy(table_hbm.at[idx_v.at[nxt]],
                                 rows_v.at[nxt % NBUF], gsem)
            )
    for j in range(NBUF):
        writes[(NCHUNK - NBUF + j) % NBUF].wait()


_gather_call = functools.partial(
    pl.kernel,
    out_type=jax.ShapeDtypeStruct((TOTAL_ROWS, HIDDEN), jnp.int32),
    mesh=plsc.VectorSubcoreMesh(core_axis_name="c", subcore_axis_name="s"),
    scratch_types=[
        pltpu.VMEM((NCHUNK, CHUNK), jnp.int32),
        pltpu.VMEM((NBUF, CHUNK, HIDDEN), jnp.int32),
        pltpu.SemaphoreType.DMA,
        pltpu.SemaphoreType.DMA,
    ],
    compiler_params=pltpu.CompilerParams(use_tc_tiling_on_sc=True),
)(_gather_body)


def _select_row(w, b_hi, b_half):
    low = lax.bitcast_convert_type(w << 16, jnp.float32)
    high = lax.bitcast_convert_type(w & jnp.int32(-65536), jnp.float32)
    row = low + b_hi * (high - low)
    return row[:, :EMBED] + b_half * (row[:, EMBED:] - row[:, :EMBED])


def _mlp_body(g0_ref, g1_ref, s0_ref, s1_ref, w1_ref, b1_ref, w2_ref,
              b2_ref, o_ref):
    s0 = s0_ref[...]
    s1 = s1_ref[...]
    x0 = _select_row(g0_ref[...], s0[:, 1:2], s0[:, 0:1])
    x1 = _select_row(g1_ref[...], s1[:, 1:2], s1[:, 0:1])
    w1 = w1_ref[...]
    h = lax.dot_general(
        x0, w1[:, :EMBED], (((1,), (1,)), ((), ())),
        preferred_element_type=jnp.float32,
    )
    h = h + lax.dot_general(
        x1, w1[:, EMBED:], (((1,), (1,)), ((), ())),
        preferred_element_type=jnp.float32,
    )
    h = jnp.maximum(h + b1_ref[...], 0.0)
    z = lax.dot_general(
        h, w2_ref[...], (((1,), (0,)), ((), ())),
        preferred_element_type=jnp.float32,
    )
    o_ref[...] = jax.nn.sigmoid(z + b2_ref[0, 0])


def _mlp_call(g, sub, W1, b1, W2, b2):
    blk = 4096
    nblk = BATCH // blk
    return pl.pallas_call(
        _mlp_body,
        grid=(nblk,),
        in_specs=[
            pl.BlockSpec((blk, HIDDEN), lambda i: (i, 0)),
            pl.BlockSpec((blk, HIDDEN), lambda i: (nblk + i, 0)),
            pl.BlockSpec((blk, 2), lambda i: (i, 0)),
            pl.BlockSpec((blk, 2), lambda i: (nblk + i, 0)),
            pl.BlockSpec((HIDDEN, HIDDEN), lambda i: (0, 0)),
            pl.BlockSpec((1, HIDDEN), lambda i: (0, 0)),
            pl.BlockSpec((HIDDEN, 1), lambda i: (0, 0)),
            pl.BlockSpec((1, 1), lambda i: (0, 0)),
        ],
        out_specs=pl.BlockSpec((blk, 1), lambda i: (i, 0)),
        out_shape=jax.ShapeDtypeStruct((BATCH, 1), jnp.float32),
    )(g, g, sub, sub, W1, b1, W2, b2)


def _bf16bits(x):
    b = lax.bitcast_convert_type(x, jnp.int32)
    return ((b + jnp.int32(0x7FFF) + ((b >> 16) & jnp.int32(1))) >> 16) \
        & jnp.int32(0xFFFF)


VBLK = 32768


def _pack_body(tt_ref, o_ref):
    eye = jnp.bfloat16(
        lax.broadcasted_iota(jnp.int32, (EMBED, EMBED), 0)
        == lax.broadcasted_iota(jnp.int32, (EMBED, EMBED), 1))
    xt = lax.dot_general(
        tt_ref[...].astype(jnp.bfloat16), eye, (((0,), (0,)), ((), ())),
        preferred_element_type=jnp.float32)
    q4 = VBLK // 4
    lo = lax.bitcast_convert_type(
        jnp.concatenate([xt[:q4], xt[q4:2 * q4]], axis=1), jnp.int32)
    hi = lax.bitcast_convert_type(
        jnp.concatenate([xt[2 * q4:3 * q4], xt[3 * q4:]], axis=1), jnp.int32)
    o_ref[...] = ((lo >> 16) & jnp.int32(0xFFFF)) | (hi & jnp.int32(-65536))


NPACK = -(-VOCAB // VBLK)


def _pack_call(tt):
    return pl.pallas_call(
        _pack_body,
        grid=(NPACK,),
        in_specs=[pl.BlockSpec((EMBED, VBLK), lambda i: (0, i))],
        out_specs=pl.BlockSpec((VBLK // 4, 2 * EMBED), lambda i: (i, 0)),
        out_shape=jax.ShapeDtypeStruct((NPACK * VBLK // 4, 2 * EMBED),
                                       jnp.int32),
    )(tt)


@jax.jit
def kernel(pairs, table, W1, b1, W2, b2):
    t32 = _pack_call(table.T)
    flat = pairs.T.reshape(TOTAL_ROWS)
    q = (flat % VBLK) // (VBLK // 4)
    row = (flat // VBLK) * (VBLK // 4) + flat % (VBLK // 4)
    idx = row.reshape(NW, NCHUNK, CHUNK)
    sub = jnp.stack(
        [(q & 1).astype(jnp.float32),
         (q >> 1).astype(jnp.float32)], axis=1)
    g = _gather_call(t32, idx)
    return _mlp_call(g, sub, W1, b1.reshape(1, HIDDEN),
                     W2.reshape(HIDDEN, 1), b2.reshape(1, 1))


# speedup vs baseline: 2.9437x; 1.0069x over previous
import functools

import jax
import jax.numpy as jnp
from jax import lax
from jax.experimental import pallas as pl
from jax.experimental.pallas import tpu as pltpu
from jax.experimental.pallas import tpu_sc as plsc

VOCAB = 1000000
EMBED = 64
HIDDEN = 128
BATCH = 16384

NC, NS = 2, 16
NW = NC * NS
TOTAL_ROWS = 2 * BATCH
ROWS_PER_W = TOTAL_ROWS // NW
CHUNK = 128
NCHUNK = ROWS_PER_W // CHUNK
NBUF = 4

VBLK = 32768
NPACK = -(-VOCAB // VBLK)


def _gather_body(table_hbm, idx_hbm, out_hbm, idx_v, rows_v, gsem, wsem):
    wid = lax.axis_index("s") * NC + lax.axis_index("c")
    base = wid * ROWS_PER_W
    pltpu.sync_copy(idx_hbm.at[wid], idx_v)
    gathers = []
    writes = [None] * NBUF
    for j in range(NBUF):
        gathers.append(
            pltpu.async_copy(table_hbm.at[idx_v.at[j]], rows_v.at[j], gsem)
        )
    for j in range(NCHUNK):
        gathers[j].wait()
        writes[j % NBUF] = pltpu.async_copy(
            rows_v.at[j % NBUF],
            out_hbm.at[pl.ds(base + j * CHUNK, CHUNK)],
            wsem,
        )
        nxt = j + NBUF
        if nxt < NCHUNK:
            writes[nxt % NBUF].wait()
            gathers.append(
                pltpu.async_copy(table_hbm.at[idx_v.at[nxt]],
                                 rows_v.at[nxt % NBUF], gsem)
            )
    for j in range(NBUF):
        writes[(NCHUNK - NBUF + j) % NBUF].wait()


_gather_call = functools.partial(
    pl.kernel,
    out_type=jax.ShapeDtypeStruct((TOTAL_ROWS, HIDDEN), jnp.int32),
    mesh=plsc.VectorSubcoreMesh(core_axis_name="c", subcore_axis_name="s"),
    scratch_types=[
        pltpu.VMEM((NCHUNK, CHUNK), jnp.int32),
        pltpu.VMEM((NBUF, CHUNK, HIDDEN), jnp.int32),
        pltpu.SemaphoreType.DMA,
        pltpu.SemaphoreType.DMA,
    ],
    compiler_params=pltpu.CompilerParams(use_tc_tiling_on_sc=True),
)(_gather_body)


def _select_row(w, b_hi, b_half):
    low = lax.bitcast_convert_type(w << 16, jnp.float32)
    high = lax.bitcast_convert_type(w & jnp.int32(-65536), jnp.float32)
    row = low + b_hi * (high - low)
    return row[:, :EMBED] + b_half * (row[:, EMBED:] - row[:, :EMBED])


def _mlp_body(g0_ref, g1_ref, s0_ref, s1_ref, w1_ref, b1_ref, w2_ref,
              b2_ref, o_ref):
    s0 = s0_ref[...]
    s1 = s1_ref[...]
    x0 = _select_row(g0_ref[...], s0[:, 1:2], s0[:, 0:1])
    x1 = _select_row(g1_ref[...], s1[:, 1:2], s1[:, 0:1])
    w1 = w1_ref[...]
    h = lax.dot_general(
        x0, w1[:, :EMBED], (((1,), (1,)), ((), ())),
        preferred_element_type=jnp.float32,
    )
    h = h + lax.dot_general(
        x1, w1[:, EMBED:], (((1,), (1,)), ((), ())),
        preferred_element_type=jnp.float32,
    )
    h = jnp.maximum(h + b1_ref[...], 0.0)
    z = lax.dot_general(
        h, w2_ref[...], (((1,), (0,)), ((), ())),
        preferred_element_type=jnp.float32,
    )
    o_ref[...] = jax.nn.sigmoid(z + b2_ref[0, 0])


def _mlp_call(g, sub, W1, b1, W2, b2):
    blk = 4096
    nblk = BATCH // blk
    return pl.pallas_call(
        _mlp_body,
        grid=(nblk,),
        in_specs=[
            pl.BlockSpec((blk, HIDDEN), lambda i: (i, 0)),
            pl.BlockSpec((blk, HIDDEN), lambda i: (nblk + i, 0)),
            pl.BlockSpec((blk, 2), lambda i: (i, 0)),
            pl.BlockSpec((blk, 2), lambda i: (nblk + i, 0)),
            pl.BlockSpec((HIDDEN, HIDDEN), lambda i: (0, 0)),
            pl.BlockSpec((1, HIDDEN), lambda i: (0, 0)),
            pl.BlockSpec((HIDDEN, 1), lambda i: (0, 0)),
            pl.BlockSpec((1, 1), lambda i: (0, 0)),
        ],
        out_specs=pl.BlockSpec((blk, 1), lambda i: (i, 0)),
        out_shape=jax.ShapeDtypeStruct((BATCH, 1), jnp.float32),
    )(g, g, sub, sub, W1, b1, W2, b2)


def _pack_body(tt_ref, o_ref):
    eye = jnp.bfloat16(
        lax.broadcasted_iota(jnp.int32, (EMBED, EMBED), 0)
        == lax.broadcasted_iota(jnp.int32, (EMBED, EMBED), 1))
    xt = lax.dot_general(
        tt_ref[...].astype(jnp.bfloat16), eye, (((0,), (0,)), ((), ())),
        preferred_element_type=jnp.float32)
    q4 = VBLK // 4
    lo = lax.bitcast_convert_type(
        jnp.concatenate([xt[:q4], xt[q4:2 * q4]], axis=1), jnp.int32)
    hi = lax.bitcast_convert_type(
        jnp.concatenate([xt[2 * q4:3 * q4], xt[3 * q4:]], axis=1), jnp.int32)
    o_ref[...] = ((lo >> 16) & jnp.int32(0xFFFF)) | (hi & jnp.int32(-65536))


def _pack_call(tt):
    return pl.pallas_call(
        _pack_body,
        grid=(NPACK,),
        in_specs=[pl.BlockSpec((EMBED, VBLK), lambda i: (0, i))],
        out_specs=pl.BlockSpec((VBLK // 4, 2 * EMBED), lambda i: (i, 0)),
        out_shape=jax.ShapeDtypeStruct((NPACK * VBLK // 4, 2 * EMBED),
                                       jnp.int32),
    )(tt)


@jax.jit
def kernel(pairs, table, W1, b1, W2, b2):
    t32 = _pack_call(table.T)
    flat = pairs.T.reshape(TOTAL_ROWS)
    q = (flat % VBLK) // (VBLK // 4)
    row = (flat // VBLK) * (VBLK // 4) + flat % (VBLK // 4)
    idx = row.reshape(NW, NCHUNK, CHUNK)
    sub = jnp.stack(
        [(q & 1).astype(jnp.float32),
         (q >> 1).astype(jnp.float32)], axis=1)
    g = _gather_call(t32, idx)
    return _mlp_call(g, sub, W1, b1.reshape(1, HIDDEN),
                     W2.reshape(HIDDEN, 1), b2.reshape(1, 1))
